# baseline (speedup 1.0000x reference)
import functools

import jax
import jax.numpy as jnp
from jax import lax
from jax.experimental import pallas as pl
from jax.experimental.pallas import tpu as pltpu
from jax.experimental.pallas import tpu_sc as plsc


@functools.lru_cache(maxsize=None)
def _build(V, D, B):
    info = plsc.get_sparse_core_info()
    NC, NS = info.num_cores, info.num_subcores
    NW = NC * NS
    b_per_w = B // NW
    CH = 128
    n_ch = b_per_w // CH
    mesh = plsc.VectorSubcoreMesh(core_axis_name="c", subcore_axis_name="s")

    @functools.partial(
        pl.kernel,
        mesh=mesh,
        out_type=jax.ShapeDtypeStruct((2, B, D), jnp.float32),
        scratch_types=[
            pltpu.VMEM((2, b_per_w), jnp.int32),
            pltpu.VMEM((2, b_per_w, D), jnp.float32),
            pltpu.SemaphoreType.DMA,
            pltpu.SemaphoreType.DMA,
        ],
        compiler_params=pltpu.CompilerParams(use_tc_tiling_on_sc=False),
    )
    def k(wt_hbm, ct_hbm, words_hbm, ctx_hbm, out_hbm, idx_v, rows_v, sem0, sem1):
        wid = lax.axis_index("s") * NC + lax.axis_index("c")
        base = wid * b_per_w

        pltpu.sync_copy(words_hbm.at[pl.ds(base, b_per_w)], idx_v.at[0])
        pltpu.sync_copy(ctx_hbm.at[pl.ds(base, b_per_w)], idx_v.at[1])

        tables = (wt_hbm, ct_hbm)
        sems = (sem0, sem1)
        copies = [[], []]
        for t in range(2):
            for c in range(n_ch):
                copies[t].append(
                    pltpu.async_copy(
                        tables[t].at[idx_v.at[t, pl.ds(c * CH, CH)]],
                        rows_v.at[t, pl.ds(c * CH, CH), :],
                        sems[t],
                    )
                )
        for t in range(2):
            for cp in copies[t]:
                cp.wait()
            pltpu.sync_copy(rows_v.at[t], out_hbm.at[t, pl.ds(base, b_per_w), :])

    return k


def kernel(words, contexts, w_table, c_table):
    V, D = w_table.shape
    B = words.shape[0]
    k = _build(V, D, B)
    return k(w_table, c_table, words, contexts)


# speedup vs baseline: 2.4216x; 2.4216x over previous
import functools

import jax
import jax.numpy as jnp
from jax import lax
from jax.experimental import pallas as pl
from jax.experimental.pallas import tpu as pltpu
from jax.experimental.pallas import tpu_sc as plsc

_LANES = 16
_RING = 4


@functools.lru_cache(maxsize=None)
def _build(V, D, B):
    info = plsc.get_sparse_core_info()
    NC, NS = info.num_cores, info.num_subcores
    NW = NC * NS
    b_per_w = B // NW
    mesh = plsc.VectorSubcoreMesh(core_axis_name="c", subcore_axis_name="s")
    n_chunk = D // _LANES

    @functools.partial(
        pl.kernel,
        mesh=mesh,
        out_type=jax.ShapeDtypeStruct((2, D, B), jnp.float32),
        scratch_types=[
            pltpu.SMEM((2, b_per_w), jnp.int32),
            pltpu.VMEM((2 * b_per_w,), jnp.int32),
            pltpu.VMEM((_RING, D, 128), jnp.float32),
            pltpu.VMEM((2, D, b_per_w), jnp.float32),
            pltpu.SemaphoreType.DMA((_RING,)),
            pltpu.SemaphoreType.DMA,
        ],
        compiler_params=pltpu.CompilerParams(needs_layout_passes=False),
    )
    def k(wtT_hbm, ctT_hbm, words_hbm, ctx_hbm, out_hbm, idx_s, idx_v,
          blk_v, rows_v, ring_sem, out_sem):
        wid = lax.axis_index("s") * NC + lax.axis_index("c")
        base = wid * b_per_w

        pltpu.sync_copy(words_hbm.at[pl.ds(base, b_per_w)],
                        idx_v.at[pl.ds(0, b_per_w)])
        pltpu.sync_copy(ctx_hbm.at[pl.ds(base, b_per_w)],
                        idx_v.at[pl.ds(b_per_w, b_per_w)])

        tables = (wtT_hbm, ctT_hbm)
        lane = lax.iota(jnp.int32, _LANES)

        for t in range(2):
            def stage(g, t=t):
                vec = plsc.load_gather(
                    idx_v, [t * b_per_w + g * _LANES + lane]
                )
                for l in range(_LANES):
                    s = lax.reduce_max(
                        jnp.where(lane == l, vec, jnp.int32(-2147483648)),
                        axes=(0,),
                    )
                    idx_s[t, g * _LANES + l] = s
            pl.loop(0, b_per_w // _LANES)(stage)

        def fire(t, j, slot):
            i = jnp.clip(idx_s[t, j], 0, V - 1)
            blk = pl.multiple_of((i >> 7) * 128, 128)
            return pltpu.async_copy(
                tables[t].at[:, pl.ds(blk, 128)],
                blk_v.at[slot],
                ring_sem.at[slot],
            )

        def extract(t, j, slot):
            i = idx_s[t, j]
            col = jnp.full((_LANES,), i & 127, dtype=jnp.int32)
            jcol = jnp.full((_LANES,), j, dtype=jnp.int32)
            for c in range(n_chunk):
                dvec = lane + c * _LANES
                vals = plsc.load_gather(blk_v.at[slot], [dvec, col])
                plsc.store_scatter(rows_v.at[t], [dvec, jcol], vals)

        n_grp = b_per_w // _RING
        for t in range(2):
            for r in range(_RING):
                fire(t, r, r)

            def grp(g, t=t):
                for r in range(_RING):
                    j = g * _RING + r
                    pltpu.make_async_copy(
                        tables[t].at[:, pl.ds(0, 128)],
                        blk_v.at[r],
                        ring_sem.at[r],
                    ).wait()
                    extract(t, j, r)

                    @pl.when(g < n_grp - 1)
                    def _():
                        fire(t, j + _RING, r)

            pl.loop(0, n_grp)(grp)

            pltpu.sync_copy(
                rows_v.at[t], out_hbm.at[t, :, pl.ds(base, b_per_w)]
            )

    return k


def kernel(words, contexts, w_table, c_table):
    V, D = w_table.shape
    B = words.shape[0]
    k = _build(V, D, B)
    out_t = k(
        jnp.swapaxes(w_table, 0, 1),
        jnp.swapaxes(c_table, 0, 1),
        words,
        contexts,
    )
    return jnp.swapaxes(out_t, 1, 2)


# speedup vs baseline: 2.8476x; 1.1759x over previous
import functools

import jax
import jax.numpy as jnp
from jax import lax
from jax.experimental import pallas as pl
from jax.experimental.pallas import tpu as pltpu
from jax.experimental.pallas import tpu_sc as plsc

_LANES = 16
_RING = 8


@functools.lru_cache(maxsize=None)
def _build(V, D, B):
    info = plsc.get_sparse_core_info()
    NC, NS = info.num_cores, info.num_subcores
    NW = NC * NS
    b_per_w = B // NW
    mesh = plsc.VectorSubcoreMesh(core_axis_name="c", subcore_axis_name="s")
    n_chunk = D // _LANES

    @functools.partial(
        pl.kernel,
        mesh=mesh,
        out_type=jax.ShapeDtypeStruct((2, D, B), jnp.float32),
        scratch_types=[
            pltpu.SMEM((2, b_per_w), jnp.int32),
            pltpu.VMEM((2 * b_per_w,), jnp.int32),
            pltpu.VMEM((_RING, D, 128), jnp.float32),
            pltpu.VMEM((D, b_per_w), jnp.float32),
            pltpu.SemaphoreType.DMA((_RING,)),
            pltpu.SemaphoreType.DMA,
        ],
        compiler_params=pltpu.CompilerParams(needs_layout_passes=False),
    )
    def k(wtT_hbm, ctT_hbm, words_hbm, ctx_hbm, out_hbm, idx_s, idx_v,
          blk_v, rows_v, ring_sem, out_sem):
        wid = lax.axis_index("s") * NC + lax.axis_index("c")
        base = wid * b_per_w

        pltpu.sync_copy(words_hbm.at[pl.ds(base, b_per_w)],
                        idx_v.at[pl.ds(0, b_per_w)])
        pltpu.sync_copy(ctx_hbm.at[pl.ds(base, b_per_w)],
                        idx_v.at[pl.ds(b_per_w, b_per_w)])

        tables = (wtT_hbm, ctT_hbm)
        lane = lax.iota(jnp.int32, _LANES)

        for t in range(2):
            def stage(g, t=t):
                vec = plsc.load_gather(
                    idx_v, [t * b_per_w + g * _LANES + lane]
                )
                for l in range(_LANES):
                    s = lax.reduce_max(
                        jnp.where(lane == l, vec, jnp.int32(-2147483648)),
                        axes=(0,),
                    )
                    idx_s[t, g * _LANES + l] = s
            pl.loop(0, b_per_w // _LANES)(stage)

        def fire(t, j, slot):
            i = jnp.clip(idx_s[t, j], 0, V - 1)
            blk = pl.multiple_of((i >> 7) * 128, 128)
            return pltpu.async_copy(
                tables[t].at[:, pl.ds(blk, 128)],
                blk_v.at[slot],
                ring_sem.at[slot],
            )

        def extract(t, j, slot):
            i = idx_s[t, j]
            col = jnp.full((_LANES,), i & 127, dtype=jnp.int32)
            jcol = jnp.full((_LANES,), j, dtype=jnp.int32)
            for c in range(n_chunk):
                dvec = lane + c * _LANES
                vals = plsc.load_gather(blk_v.at[slot], [dvec, col])
                plsc.store_scatter(rows_v, [dvec, jcol], vals)

        n_grp = b_per_w // _RING
        for t in range(2):
            for r in range(_RING):
                fire(t, r, r)

            def grp(g, t=t):
                for r in range(_RING):
                    j = g * _RING + r
                    pltpu.make_async_copy(
                        tables[t].at[:, pl.ds(0, 128)],
                        blk_v.at[r],
                        ring_sem.at[r],
                    ).wait()
                    extract(t, j, r)

                    @pl.when(g < n_grp - 1)
                    def _():
                        fire(t, j + _RING, r)

            pl.loop(0, n_grp)(grp)

            pltpu.sync_copy(
                rows_v, out_hbm.at[t, :, pl.ds(base, b_per_w)]
            )

    return k


def kernel(words, contexts, w_table, c_table):
    V, D = w_table.shape
    B = words.shape[0]
    k = _build(V, D, B)
    out_t = k(
        jnp.swapaxes(w_table, 0, 1),
        jnp.swapaxes(c_table, 0, 1),
        words,
        contexts,
    )
    return jnp.swapaxes(out_t, 1, 2)
